# speedup vs baseline: 1.4459x; 1.4459x over previous
import functools

import jax
import jax.numpy as jnp
from jax.experimental import pallas as pl

_D = 128
_NSEG = 10000
_NORM = float(_D) ** 0.5


def _e_kernel(x_ref, mask_ref, wt_ref, ctx_ref, b_ref, mv_ref, e_ref):
    x = x_ref[...]
    h = jnp.tanh(
        jnp.dot(x, wt_ref[...], preferred_element_type=jnp.float32) + b_ref[...]
    )
    imp = jnp.dot(h, ctx_ref[...], preferred_element_type=jnp.float32) * (1.0 / _NORM)
    imp = jnp.where(mask_ref[...] == 0, mv_ref[...], imp)
    e_ref[...] = jnp.exp(imp)


@jax.jit
def kernel(input, mask, index, context, W, b):
    n = input.shape[0]
    blk = 2560
    grid = n // blk

    L = jnp.sum(jnp.abs(context)) / _NORM
    maskval = (-(L + 40.0)).reshape(1, 1)

    e = pl.pallas_call(
        _e_kernel,
        grid=(grid,),
        in_specs=[
            pl.BlockSpec((blk, _D), lambda i: (i, 0)),
            pl.BlockSpec((blk, 1), lambda i: (i, 0)),
            pl.BlockSpec((_D, _D), lambda i: (0, 0)),
            pl.BlockSpec((_D, 1), lambda i: (0, 0)),
            pl.BlockSpec((1, _D), lambda i: (0, 0)),
            pl.BlockSpec((1, 1), lambda i: (0, 0)),
        ],
        out_specs=pl.BlockSpec((blk, 1), lambda i: (i, 0)),
        out_shape=jax.ShapeDtypeStruct((n, 1), jnp.float32),
    )(
        input,
        mask.reshape(n, 1),
        W.T,
        context.reshape(_D, 1),
        b.reshape(1, _D),
        maskval,
    )
    e = e.reshape(n)

    denom = jax.ops.segment_sum(e, index, num_segments=_NSEG)
    aw = e / denom[index]
    out = jax.ops.segment_sum(input * aw[:, None], index, num_segments=_NSEG)
    return (out, aw)


# speedup vs baseline: 3.6481x; 2.5231x over previous
import functools

import jax
import jax.numpy as jnp
from jax import lax
from jax.experimental import pallas as pl
from jax.experimental.pallas import tpu as pltpu
from jax.experimental.pallas import tpu_sc as plsc

_D = 128
_NSEG = 10000
_NORM = float(_D) ** 0.5
_N = 320000
_NC = 2
_NS = 16
_NW = _NC * _NS
_CHUNK = _N // _NW
_VL = 16
_WSEG = 256
_SUBR = 100
_PIECE = 1000

_mesh = functools.partial(
    plsc.VectorSubcoreMesh,
    core_axis_name="c",
    subcore_axis_name="s",
    num_cores=_NC,
    num_subcores=_NS,
)

_SC_PARAMS = pltpu.CompilerParams(needs_layout_passes=False)


def _e_kernel(x_ref, mask_ref, wt_ref, ctx_ref, b_ref, mv_ref, e_ref):
    x = x_ref[...]
    h = jnp.tanh(
        jnp.dot(x, wt_ref[...], preferred_element_type=jnp.float32) + b_ref[...]
    )
    imp = jnp.dot(h, ctx_ref[...], preferred_element_type=jnp.float32) * (1.0 / _NORM)
    imp = jnp.where(mask_ref[...] == 0, mv_ref[...], imp)
    e_ref[...] = jnp.exp(imp)


def _wid():
    return lax.axis_index("s") * _NC + lax.axis_index("c")


def _sc_denom_partials(e_hbm, idx_hbm, out_hbm, e_v, idx_v, acc_v):
    wid = _wid()
    base = wid * _CHUNK
    pltpu.sync_copy(e_hbm.at[pl.ds(base, _CHUNK)], e_v)
    pltpu.sync_copy(idx_hbm.at[pl.ds(base, _CHUNK)], idx_v)

    def zero_body(i, carry):
        acc_v[pl.ds(i * _VL, _VL)] = jnp.zeros((_VL,), jnp.float32)
        return carry

    lax.fori_loop(0, _NSEG // _VL, zero_body, 0)

    def body(i, carry):
        s = pl.ds(i * _VL, _VL)
        plsc.addupdate_scatter(acc_v, [idx_v[s]], e_v[s])
        return carry

    lax.fori_loop(0, _CHUNK // _VL, body, 0)
    pltpu.sync_copy(acc_v, out_hbm.at[pl.ds(wid * _NSEG, _NSEG)])


def _sc_aw(part_hbm, e_hbm, idx_hbm, aw_hbm, den_hbm, den_v, buf_v, e_v, idx_v, aw_v):
    wid = _wid()
    base = wid * _CHUNK
    pltpu.sync_copy(part_hbm.at[pl.ds(0, _NSEG)], den_v)

    def outer(j, carry):
        pltpu.sync_copy(part_hbm.at[pl.ds(j * _NSEG, _NSEG)], buf_v)

        def inner(i, c2):
            s = pl.ds(i * _VL, _VL)
            den_v[s] = den_v[s] + buf_v[s]
            return c2

        return lax.fori_loop(0, _NSEG // _VL, inner, carry)

    lax.fori_loop(1, _NW, outer, 0)

    pltpu.sync_copy(e_hbm.at[pl.ds(base, _CHUNK)], e_v)
    pltpu.sync_copy(idx_hbm.at[pl.ds(base, _CHUNK)], idx_v)

    def body(i, carry):
        s = pl.ds(i * _VL, _VL)
        d = plsc.load_gather(den_v, [idx_v[s]])
        aw_v[s] = e_v[s] / d
        return carry

    lax.fori_loop(0, _CHUNK // _VL, body, 0)
    pltpu.sync_copy(aw_v, aw_hbm.at[pl.ds(base, _CHUNK)])

    @pl.when(wid == 0)
    def _():
        pltpu.sync_copy(den_v, den_hbm)


def _sc_wsum(
    x_hbm, e_hbm, idx_hbm, out_hbm,
    x_buf, e_v, idx_v, acc, f0, f1, shared,
):
    wid = _wid()
    cid = lax.axis_index("c")
    sub = lax.axis_index("s")
    base = wid * _CHUNK
    col0 = lax.iota(jnp.int32, _VL)
    zvec = jnp.zeros((_VL,), jnp.float32)
    fidx = (f0, f1)

    def zero_acc():
        def zb(j, c):
            rvec = jnp.full((_VL,), j, jnp.int32)
            for k in range(8):
                plsc.store_scatter(acc, [rvec, col0 + k * _VL], zvec)
            return c

        lax.fori_loop(0, _WSEG, zb, 0)

    def flush(first):
        fvec = jnp.full((_VL,), first, jnp.int32)
        for q in range(2):
            for j in range(8):
                v = fvec + (q * 128 + j * _VL) + col0
                v = jnp.where(v < _NSEG, v, _NSEG)
                fidx[q][pl.ds(j * _VL, _VL)] = v
            pltpu.sync_copy(acc.at[pl.ds(q * 128, 128)], shared.at[fidx[q]], add=True)
        zero_acc()

    zero_acc()

    @pl.when(sub < 15)
    def _():
        pltpu.sync_copy(acc.at[pl.ds(0, 512)], shared.at[pl.ds(sub * 632, 512)])
        pltpu.sync_copy(acc.at[pl.ds(0, 120)], shared.at[pl.ds(sub * 632 + 512, 120)])

    @pl.when(sub == 15)
    def _():
        pltpu.sync_copy(acc.at[pl.ds(0, 512)], shared.at[pl.ds(9480, 512)])
        pltpu.sync_copy(acc.at[pl.ds(0, 9)], shared.at[pl.ds(9992, 9)])

    plsc.subcore_barrier()

    def piece_body(p, first):
        pltpu.sync_copy(idx_hbm.at[pl.ds(base + p * _PIECE, _PIECE)], idx_v)
        pltpu.sync_copy(e_hbm.at[pl.ds(base + p * _PIECE, _PIECE)], e_v)

        def sub_body(q, fst0):
            row0 = p * _PIECE + q * _SUBR
            pltpu.sync_copy(
                x_hbm.at[pl.ds((base + row0) * _D, _SUBR * _D)], x_buf
            )

            def row_body(r, fst):
                rs = jnp.full((_VL,), q * _SUBR + r, jnp.int32)
                seg = plsc.load_gather(idx_v, [rs])
                segs = jnp.max(seg)
                need = (segs - fst) >= _WSEG
                pl.when(need)(lambda: flush(fst))
                fst = jnp.where(need, segs, fst)
                ew = plsc.load_gather(e_v, [rs])
                loff = seg - jnp.full((_VL,), fst, jnp.int32)
                for k in range(8):
                    xv = x_buf[pl.ds(r * _D + k * _VL, _VL)]
                    plsc.addupdate_scatter(acc, [loff, col0 + k * _VL], xv * ew)
                return fst

            return lax.fori_loop(0, _SUBR, row_body, fst0)

        return lax.fori_loop(0, _PIECE // _SUBR, sub_body, first)

    first_end = lax.fori_loop(0, _CHUNK // _PIECE, piece_body, 0)
    flush(first_end)
    plsc.subcore_barrier()

    @pl.when(sub < 15)
    def _():
        pltpu.sync_copy(
            shared.at[pl.ds(sub * 632, 632)],
            out_hbm.at[pl.ds(cid * _NSEG + sub * 632, 632)],
        )

    @pl.when(sub == 15)
    def _():
        pltpu.sync_copy(
            shared.at[pl.ds(9480, 520)],
            out_hbm.at[pl.ds(cid * _NSEG + 9480, 520)],
        )


def _tc3_kernel(p_ref, d_ref, o_ref):
    s = p_ref[0] + p_ref[1]
    o_ref[...] = s / jnp.maximum(d_ref[...], 1e-30)


@jax.jit
def kernel(input, mask, index, context, W, b):
    n = input.shape[0]
    blk = 2560
    grid = n // blk

    L = jnp.sum(jnp.abs(context)) / _NORM
    maskval = (-(L + 40.0)).reshape(1, 1)

    e = pl.pallas_call(
        _e_kernel,
        grid=(grid,),
        in_specs=[
            pl.BlockSpec((blk, _D), lambda i: (i, 0)),
            pl.BlockSpec((blk, 1), lambda i: (i, 0)),
            pl.BlockSpec((_D, _D), lambda i: (0, 0)),
            pl.BlockSpec((_D, 1), lambda i: (0, 0)),
            pl.BlockSpec((1, _D), lambda i: (0, 0)),
            pl.BlockSpec((1, 1), lambda i: (0, 0)),
        ],
        out_specs=pl.BlockSpec((blk, 1), lambda i: (i, 0)),
        out_shape=jax.ShapeDtypeStruct((n, 1), jnp.float32),
    )(
        input,
        mask.reshape(n, 1),
        W.T,
        context.reshape(_D, 1),
        b.reshape(1, _D),
        maskval,
    )
    e = e.reshape(n)

    partials = pl.kernel(
        _sc_denom_partials,
        out_type=jax.ShapeDtypeStruct((_NW * _NSEG,), jnp.float32),
        mesh=_mesh(),
        compiler_params=_SC_PARAMS,
        scratch_types=[
            pltpu.VMEM((_CHUNK,), jnp.float32),
            pltpu.VMEM((_CHUNK,), jnp.int32),
            pltpu.VMEM((_NSEG,), jnp.float32),
        ],
    )(e, index)

    aw, den = pl.kernel(
        _sc_aw,
        out_type=(
            jax.ShapeDtypeStruct((n,), jnp.float32),
            jax.ShapeDtypeStruct((_NSEG,), jnp.float32),
        ),
        mesh=_mesh(),
        compiler_params=_SC_PARAMS,
        scratch_types=[
            pltpu.VMEM((_NSEG,), jnp.float32),
            pltpu.VMEM((_NSEG,), jnp.float32),
            pltpu.VMEM((_CHUNK,), jnp.float32),
            pltpu.VMEM((_CHUNK,), jnp.int32),
            pltpu.VMEM((_CHUNK,), jnp.float32),
        ],
    )(partials, e, index)

    pooled = pl.kernel(
        _sc_wsum,
        out_type=jax.ShapeDtypeStruct((_NC * _NSEG, _D), jnp.float32),
        mesh=_mesh(),
        compiler_params=_SC_PARAMS,
        scratch_types=[
            pltpu.VMEM((_SUBR * _D,), jnp.float32),
            pltpu.VMEM((_PIECE,), jnp.float32),
            pltpu.VMEM((_PIECE,), jnp.int32),
            pltpu.VMEM((_WSEG, _D), jnp.float32),
            pltpu.VMEM((128,), jnp.int32),
            pltpu.VMEM((128,), jnp.int32),
            pltpu.VMEM_SHARED((_NSEG + 1, _D), jnp.float32),
        ],
    )(input.reshape(n * _D), e, index)

    out = pl.pallas_call(
        _tc3_kernel,
        grid=(5,),
        in_specs=[
            pl.BlockSpec((2, 2000, _D), lambda i: (0, i, 0)),
            pl.BlockSpec((2000, 1), lambda i: (i, 0)),
        ],
        out_specs=pl.BlockSpec((2000, _D), lambda i: (i, 0)),
        out_shape=jax.ShapeDtypeStruct((_NSEG, _D), jnp.float32),
    )(pooled.reshape(_NC, _NSEG, _D), den.reshape(_NSEG, 1))

    return (out, aw)


# speedup vs baseline: 4.3752x; 1.1993x over previous
import functools

import jax
import jax.numpy as jnp
from jax import lax
from jax.experimental import pallas as pl
from jax.experimental.pallas import tpu as pltpu
from jax.experimental.pallas import tpu_sc as plsc

_D = 128
_NSEG = 10000
_NORM = float(_D) ** 0.5
_N = 320000
_NC = 2
_NS = 16
_NW = _NC * _NS
_CHUNK = _N // _NW
_VL = 16
_WSEG = 256
_SUBR = 100
_PIECE = 1000

_mesh = functools.partial(
    plsc.VectorSubcoreMesh,
    core_axis_name="c",
    subcore_axis_name="s",
    num_cores=_NC,
    num_subcores=_NS,
)

_SC_PARAMS = pltpu.CompilerParams(needs_layout_passes=False)


def _e_kernel(x_ref, mask_ref, wt_ref, ctx_ref, b_ref, mv_ref, e_ref):
    x = x_ref[...]
    h = jnp.tanh(
        jnp.dot(x, wt_ref[...], preferred_element_type=jnp.float32) + b_ref[...]
    )
    imp = jnp.dot(h, ctx_ref[...], preferred_element_type=jnp.float32) * (1.0 / _NORM)
    imp = jnp.where(mask_ref[...] == 0, mv_ref[...], imp)
    e_ref[...] = jnp.exp(imp)


def _wid():
    return lax.axis_index("s") * _NC + lax.axis_index("c")


def _sc_denom_partials(e_hbm, idx_hbm, out_hbm, e_v, idx_v, acc_v):
    wid = _wid()
    base = wid * _CHUNK
    pltpu.sync_copy(e_hbm.at[pl.ds(base, _CHUNK)], e_v)
    pltpu.sync_copy(idx_hbm.at[pl.ds(base, _CHUNK)], idx_v)

    def zero_body(i, carry):
        acc_v[pl.ds(i * _VL, _VL)] = jnp.zeros((_VL,), jnp.float32)
        return carry

    lax.fori_loop(0, (_NSEG + _VL) // _VL, zero_body, 0)

    lanes = lax.iota(jnp.int32, _VL)

    def _lperm(v, src):
        return lax.gather(
            v,
            src[:, None],
            lax.GatherDimensionNumbers(
                offset_dims=(), collapsed_slice_dims=(0,), start_index_map=(0,)
            ),
            slice_sizes=(1,),
            mode=lax.GatherScatterMode.PROMISE_IN_BOUNDS,
        )

    def body(i, carry):
        s = pl.ds(i * _VL, _VL)
        seg = idx_v[s]
        c = e_v[s]
        for d in (1, 2, 4, 8):
            src = jnp.maximum(lanes - d, 0)
            cs = _lperm(c, src)
            ks = _lperm(seg, src)
            ok = jnp.logical_and(lanes >= d, seg == ks)
            c = c + jnp.where(ok, cs, 0.0)
        nxt = _lperm(seg, jnp.minimum(lanes + 1, _VL - 1))
        last = jnp.logical_or(lanes == _VL - 1, seg != nxt)
        plsc.addupdate_scatter(acc_v, [seg], e_v[s])
        _unused = (c, last)
        return carry

    lax.fori_loop(0, _CHUNK // _VL, body, 0)
    pltpu.sync_copy(acc_v.at[pl.ds(0, _NSEG)], out_hbm.at[pl.ds(wid * _NSEG, _NSEG)])


def _sc_aw(part_hbm, e_hbm, idx_hbm, aw_hbm, den_hbm, den_v, buf_v, e_v, idx_v, aw_v):
    wid = _wid()
    base = wid * _CHUNK
    pltpu.sync_copy(part_hbm.at[pl.ds(0, _NSEG)], den_v)

    def outer(j, carry):
        pltpu.sync_copy(part_hbm.at[pl.ds(j * _NSEG, _NSEG)], buf_v)

        def inner(i, c2):
            s = pl.ds(i * _VL, _VL)
            den_v[s] = den_v[s] + buf_v[s]
            return c2

        return lax.fori_loop(0, _NSEG // _VL, inner, carry)

    lax.fori_loop(1, _NW, outer, 0)

    pltpu.sync_copy(e_hbm.at[pl.ds(base, _CHUNK)], e_v)
    pltpu.sync_copy(idx_hbm.at[pl.ds(base, _CHUNK)], idx_v)

    def body(i, carry):
        s = pl.ds(i * _VL, _VL)
        d = plsc.load_gather(den_v, [idx_v[s]])
        aw_v[s] = e_v[s] / d
        return carry

    lax.fori_loop(0, _CHUNK // _VL, body, 0)
    pltpu.sync_copy(aw_v, aw_hbm.at[pl.ds(base, _CHUNK)])

    @pl.when(wid == 0)
    def _():
        pltpu.sync_copy(den_v, den_hbm)


def _sc_wsum(
    x_hbm, e_hbm, idx_hbm, out_hbm,
    x_buf, e_v, idx_v, acc, f0, f1, shared,
):
    wid = _wid()
    cid = lax.axis_index("c")
    sub = lax.axis_index("s")
    base = wid * _CHUNK
    col0 = lax.iota(jnp.int32, _VL)
    zvec = jnp.zeros((_VL,), jnp.float32)
    fidx = (f0, f1)

    def zero_acc():
        def zb(j, c):
            rvec = jnp.full((_VL,), j, jnp.int32)
            for k in range(8):
                plsc.store_scatter(acc, [rvec, col0 + k * _VL], zvec)
            return c

        lax.fori_loop(0, _WSEG, zb, 0)

    def flush(first):
        fvec = jnp.full((_VL,), first, jnp.int32)
        for q in range(2):
            for j in range(8):
                v = fvec + (q * 128 + j * _VL) + col0
                v = jnp.where(v < _NSEG, v, _NSEG)
                fidx[q][pl.ds(j * _VL, _VL)] = v
            pltpu.sync_copy(acc.at[pl.ds(q * 128, 128)], shared.at[fidx[q]], add=True)
        zero_acc()

    zero_acc()

    @pl.when(sub < 15)
    def _():
        pltpu.sync_copy(acc.at[pl.ds(0, 512)], shared.at[pl.ds(sub * 632, 512)])
        pltpu.sync_copy(acc.at[pl.ds(0, 120)], shared.at[pl.ds(sub * 632 + 512, 120)])

    @pl.when(sub == 15)
    def _():
        pltpu.sync_copy(acc.at[pl.ds(0, 512)], shared.at[pl.ds(9480, 512)])
        pltpu.sync_copy(acc.at[pl.ds(0, 9)], shared.at[pl.ds(9992, 9)])

    plsc.subcore_barrier()

    def store_regs(fst, cur, regs):
        lofo = jnp.full((_VL,), cur - fst, jnp.int32)
        for k in range(8):
            plsc.addupdate_scatter(acc, [lofo, col0 + k * _VL], regs[k])

    def piece_body(p, carry):
        pltpu.sync_copy(idx_hbm.at[pl.ds(base + p * _PIECE, _PIECE)], idx_v)
        pltpu.sync_copy(e_hbm.at[pl.ds(base + p * _PIECE, _PIECE)], e_v)

        def sub_body(q, carry):
            row0 = p * _PIECE + q * _SUBR
            pltpu.sync_copy(
                x_hbm.at[pl.ds((base + row0) * _D, _SUBR * _D)], x_buf
            )

            def row_body(r, carry):
                fst, cur = carry[0], carry[1]
                regs = carry[2:]
                rs = jnp.full((_VL,), q * _SUBR + r, jnp.int32)
                seg = plsc.load_gather(idx_v, [rs])
                segs = jnp.max(seg)
                ew = plsc.load_gather(e_v, [rs])
                change = segs != cur
                pl.when(change)(lambda: store_regs(fst, cur, regs))
                need = jnp.logical_and(change, (segs - fst) >= _WSEG)
                pl.when(need)(lambda: flush(fst))
                fst = jnp.where(need, segs, fst)
                wx = [
                    x_buf[pl.ds(r * _D + k * _VL, _VL)] * ew for k in range(8)
                ]
                regs = tuple(
                    jnp.where(change, wx[k], regs[k] + wx[k]) for k in range(8)
                )
                return (fst, segs) + regs

            return lax.fori_loop(0, _SUBR, row_body, carry)

        return lax.fori_loop(0, _PIECE // _SUBR, sub_body, carry)

    zregs = tuple(jnp.zeros((_VL,), jnp.float32) for _ in range(8))
    carry0 = (jnp.int32(0), jnp.int32(0)) + zregs
    carry_end = lax.fori_loop(0, _CHUNK // _PIECE, piece_body, carry0)
    store_regs(carry_end[0], carry_end[1], carry_end[2:])
    flush(carry_end[0])
    plsc.subcore_barrier()

    @pl.when(sub < 15)
    def _():
        pltpu.sync_copy(
            shared.at[pl.ds(sub * 632, 632)],
            out_hbm.at[pl.ds(cid * _NSEG + sub * 632, 632)],
        )

    @pl.when(sub == 15)
    def _():
        pltpu.sync_copy(
            shared.at[pl.ds(9480, 520)],
            out_hbm.at[pl.ds(cid * _NSEG + 9480, 520)],
        )


def _tc3_kernel(p_ref, d_ref, o_ref):
    s = p_ref[0] + p_ref[1]
    o_ref[...] = s / jnp.maximum(d_ref[...], 1e-30)


@jax.jit
def kernel(input, mask, index, context, W, b):
    n = input.shape[0]
    blk = 2560
    grid = n // blk

    L = jnp.sum(jnp.abs(context)) / _NORM
    maskval = (-(L + 40.0)).reshape(1, 1)

    e = pl.pallas_call(
        _e_kernel,
        grid=(grid,),
        in_specs=[
            pl.BlockSpec((blk, _D), lambda i: (i, 0)),
            pl.BlockSpec((blk, 1), lambda i: (i, 0)),
            pl.BlockSpec((_D, _D), lambda i: (0, 0)),
            pl.BlockSpec((_D, 1), lambda i: (0, 0)),
            pl.BlockSpec((1, _D), lambda i: (0, 0)),
            pl.BlockSpec((1, 1), lambda i: (0, 0)),
        ],
        out_specs=pl.BlockSpec((blk, 1), lambda i: (i, 0)),
        out_shape=jax.ShapeDtypeStruct((n, 1), jnp.float32),
    )(
        input,
        mask.reshape(n, 1),
        W.T,
        context.reshape(_D, 1),
        b.reshape(1, _D),
        maskval,
    )
    e = e.reshape(n)

    partials = pl.kernel(
        _sc_denom_partials,
        out_type=jax.ShapeDtypeStruct((_NW * _NSEG,), jnp.float32),
        mesh=_mesh(),
        compiler_params=_SC_PARAMS,
        scratch_types=[
            pltpu.VMEM((_CHUNK,), jnp.float32),
            pltpu.VMEM((_CHUNK,), jnp.int32),
            pltpu.VMEM((_NSEG + _VL,), jnp.float32),
        ],
    )(e, index)

    aw, den = pl.kernel(
        _sc_aw,
        out_type=(
            jax.ShapeDtypeStruct((n,), jnp.float32),
            jax.ShapeDtypeStruct((_NSEG,), jnp.float32),
        ),
        mesh=_mesh(),
        compiler_params=_SC_PARAMS,
        scratch_types=[
            pltpu.VMEM((_NSEG,), jnp.float32),
            pltpu.VMEM((_NSEG,), jnp.float32),
            pltpu.VMEM((_CHUNK,), jnp.float32),
            pltpu.VMEM((_CHUNK,), jnp.int32),
            pltpu.VMEM((_CHUNK,), jnp.float32),
        ],
    )(partials, e, index)

    pooled = pl.kernel(
        _sc_wsum,
        out_type=jax.ShapeDtypeStruct((_NC * _NSEG, _D), jnp.float32),
        mesh=_mesh(),
        compiler_params=_SC_PARAMS,
        scratch_types=[
            pltpu.VMEM((_SUBR * _D,), jnp.float32),
            pltpu.VMEM((_PIECE,), jnp.float32),
            pltpu.VMEM((_PIECE,), jnp.int32),
            pltpu.VMEM((_WSEG, _D), jnp.float32),
            pltpu.VMEM((128,), jnp.int32),
            pltpu.VMEM((128,), jnp.int32),
            pltpu.VMEM_SHARED((_NSEG + 1, _D), jnp.float32),
        ],
    )(input.reshape(n * _D), e, index)

    out = pl.pallas_call(
        _tc3_kernel,
        grid=(5,),
        in_specs=[
            pl.BlockSpec((2, 2000, _D), lambda i: (0, i, 0)),
            pl.BlockSpec((2000, 1), lambda i: (i, 0)),
        ],
        out_specs=pl.BlockSpec((2000, _D), lambda i: (i, 0)),
        out_shape=jax.ShapeDtypeStruct((_NSEG, _D), jnp.float32),
    )(pooled.reshape(_NC, _NSEG, _D), den.reshape(_NSEG, 1))

    return (out, aw)


# speedup vs baseline: 4.3823x; 1.0016x over previous
import functools

import jax
import jax.numpy as jnp
from jax import lax
from jax.experimental import pallas as pl
from jax.experimental.pallas import tpu as pltpu
from jax.experimental.pallas import tpu_sc as plsc

_D = 128
_NSEG = 10000
_NORM = float(_D) ** 0.5
_N = 320000
_NC = 2
_NS = 16
_NW = _NC * _NS
_CHUNK = _N // _NW
_VL = 16
_WSEG = 256
_SUBR = 100
_PIECE = 1000

_mesh = functools.partial(
    plsc.VectorSubcoreMesh,
    core_axis_name="c",
    subcore_axis_name="s",
    num_cores=_NC,
    num_subcores=_NS,
)

_SC_PARAMS = pltpu.CompilerParams(needs_layout_passes=False)


def _e_kernel(x_ref, mask_ref, wt_ref, ctx_ref, b_ref, mv_ref, e_ref):
    x = x_ref[...]
    h = jnp.tanh(
        jnp.dot(x, wt_ref[...], preferred_element_type=jnp.float32) + b_ref[...]
    )
    imp = jnp.dot(h, ctx_ref[...], preferred_element_type=jnp.float32) * (1.0 / _NORM)
    imp = jnp.where(mask_ref[...] == 0, mv_ref[...], imp)
    e_ref[...] = jnp.exp(imp)


def _wid():
    return lax.axis_index("s") * _NC + lax.axis_index("c")


def _sc_denom_partials(e_hbm, idx_hbm, out_hbm, e_v, idx_v, acc_v):
    wid = _wid()
    base = wid * _CHUNK
    pltpu.sync_copy(e_hbm.at[pl.ds(base, _CHUNK)], e_v)
    pltpu.sync_copy(idx_hbm.at[pl.ds(base, _CHUNK)], idx_v)

    def zero_body(i, carry):
        acc_v[pl.ds(i * _VL, _VL)] = jnp.zeros((_VL,), jnp.float32)
        return carry

    lax.fori_loop(0, (_NSEG + _VL) // _VL, zero_body, 0)

    def body(i, carry):
        s = pl.ds(i * _VL, _VL)
        plsc.addupdate_scatter(acc_v, [idx_v[s]], e_v[s])
        return carry

    lax.fori_loop(0, _CHUNK // _VL, body, 0)
    pltpu.sync_copy(acc_v.at[pl.ds(0, _NSEG)], out_hbm.at[pl.ds(wid * _NSEG, _NSEG)])


def _sc_aw(part_hbm, e_hbm, idx_hbm, aw_hbm, den_hbm, den_v, buf_v, e_v, idx_v, aw_v):
    wid = _wid()
    base = wid * _CHUNK
    pltpu.sync_copy(part_hbm.at[pl.ds(0, _NSEG)], den_v)

    def outer(j, carry):
        pltpu.sync_copy(part_hbm.at[pl.ds(j * _NSEG, _NSEG)], buf_v)

        def inner(i, c2):
            s = pl.ds(i * _VL, _VL)
            den_v[s] = den_v[s] + buf_v[s]
            return c2

        return lax.fori_loop(0, _NSEG // _VL, inner, carry)

    lax.fori_loop(1, _NW, outer, 0)

    pltpu.sync_copy(e_hbm.at[pl.ds(base, _CHUNK)], e_v)
    pltpu.sync_copy(idx_hbm.at[pl.ds(base, _CHUNK)], idx_v)

    def body(i, carry):
        s = pl.ds(i * _VL, _VL)
        d = plsc.load_gather(den_v, [idx_v[s]])
        aw_v[s] = e_v[s] / d
        return carry

    lax.fori_loop(0, _CHUNK // _VL, body, 0)
    pltpu.sync_copy(aw_v, aw_hbm.at[pl.ds(base, _CHUNK)])

    @pl.when(wid == 0)
    def _():
        pltpu.sync_copy(den_v, den_hbm)


def _sc_wsum(
    x_hbm, e_hbm, idx_hbm, out_hbm,
    x_buf, e_v, idx_v, acc, f0, f1, shared,
):
    wid = _wid()
    cid = lax.axis_index("c")
    sub = lax.axis_index("s")
    base = wid * _CHUNK
    col0 = lax.iota(jnp.int32, _VL)
    zvec = jnp.zeros((_VL,), jnp.float32)
    fidx = (f0, f1)

    def zero_acc():
        def zb(j, c):
            rvec = jnp.full((_VL,), j, jnp.int32)
            for k in range(8):
                plsc.store_scatter(acc, [rvec, col0 + k * _VL], zvec)
            return c

        lax.fori_loop(0, _WSEG, zb, 0)

    def flush(first):
        fvec = jnp.full((_VL,), first, jnp.int32)
        for q in range(2):
            for j in range(8):
                v = fvec + (q * 128 + j * _VL) + col0
                v = jnp.where(v < _NSEG, v, _NSEG)
                fidx[q][pl.ds(j * _VL, _VL)] = v
            pltpu.sync_copy(acc.at[pl.ds(q * 128, 128)], shared.at[fidx[q]], add=True)
        zero_acc()

    zero_acc()

    @pl.when(sub < 15)
    def _():
        pltpu.sync_copy(acc.at[pl.ds(0, 512)], shared.at[pl.ds(sub * 632, 512)])
        pltpu.sync_copy(acc.at[pl.ds(0, 120)], shared.at[pl.ds(sub * 632 + 512, 120)])

    @pl.when(sub == 15)
    def _():
        pltpu.sync_copy(acc.at[pl.ds(0, 512)], shared.at[pl.ds(9480, 512)])
        pltpu.sync_copy(acc.at[pl.ds(0, 9)], shared.at[pl.ds(9992, 9)])

    plsc.subcore_barrier()

    def store_regs(fst, cur, regs):
        lofo = jnp.full((_VL,), cur - fst, jnp.int32)
        for k in range(8):
            plsc.addupdate_scatter(acc, [lofo, col0 + k * _VL], regs[k])

    def piece_body(p, carry):
        pltpu.sync_copy(idx_hbm.at[pl.ds(base + p * _PIECE, _PIECE)], idx_v)
        pltpu.sync_copy(e_hbm.at[pl.ds(base + p * _PIECE, _PIECE)], e_v)

        def sub_body(q, carry):
            row0 = p * _PIECE + q * _SUBR
            pltpu.sync_copy(
                x_hbm.at[pl.ds((base + row0) * _D, _SUBR * _D)], x_buf
            )

            def row_body(r, carry):
                fst, cur = carry[0], carry[1]
                regs = carry[2:]
                rs = jnp.full((_VL,), q * _SUBR + r, jnp.int32)
                seg = plsc.load_gather(idx_v, [rs])
                segs = jnp.max(seg)
                ew = plsc.load_gather(e_v, [rs])
                change = segs != cur
                pl.when(change)(lambda: store_regs(fst, cur, regs))
                need = jnp.logical_and(change, (segs - fst) >= _WSEG)
                pl.when(need)(lambda: flush(fst))
                fst = jnp.where(need, segs, fst)
                wx = [
                    x_buf[pl.ds(r * _D + k * _VL, _VL)] * ew for k in range(8)
                ]
                regs = tuple(
                    jnp.where(change, wx[k], regs[k] + wx[k]) for k in range(8)
                )
                return (fst, segs) + regs

            return lax.fori_loop(0, _SUBR, row_body, carry)

        return lax.fori_loop(0, _PIECE // _SUBR, sub_body, carry)

    zregs = tuple(jnp.zeros((_VL,), jnp.float32) for _ in range(8))
    carry0 = (jnp.int32(0), jnp.int32(0)) + zregs
    carry_end = lax.fori_loop(0, _CHUNK // _PIECE, piece_body, carry0)
    store_regs(carry_end[0], carry_end[1], carry_end[2:])
    flush(carry_end[0])
    plsc.subcore_barrier()

    @pl.when(sub < 15)
    def _():
        pltpu.sync_copy(
            shared.at[pl.ds(sub * 632, 632)],
            out_hbm.at[pl.ds(cid * _NSEG + sub * 632, 632)],
        )

    @pl.when(sub == 15)
    def _():
        pltpu.sync_copy(
            shared.at[pl.ds(9480, 520)],
            out_hbm.at[pl.ds(cid * _NSEG + 9480, 520)],
        )


def _tc3_kernel(p_ref, d_ref, o_ref):
    s = p_ref[0] + p_ref[1]
    o_ref[...] = s / jnp.maximum(d_ref[...], 1e-30)


@jax.jit
def kernel(input, mask, index, context, W, b):
    n = input.shape[0]
    blk = 2560
    grid = n // blk

    L = jnp.sum(jnp.abs(context)) / _NORM
    maskval = (-(L + 40.0)).reshape(1, 1)

    e = pl.pallas_call(
        _e_kernel,
        grid=(grid,),
        in_specs=[
            pl.BlockSpec((blk, _D), lambda i: (i, 0)),
            pl.BlockSpec((blk, 1), lambda i: (i, 0)),
            pl.BlockSpec((_D, _D), lambda i: (0, 0)),
            pl.BlockSpec((_D, 1), lambda i: (0, 0)),
            pl.BlockSpec((1, _D), lambda i: (0, 0)),
            pl.BlockSpec((1, 1), lambda i: (0, 0)),
        ],
        out_specs=pl.BlockSpec((blk, 1), lambda i: (i, 0)),
        out_shape=jax.ShapeDtypeStruct((n, 1), jnp.float32),
    )(
        input,
        mask.reshape(n, 1),
        W.T,
        context.reshape(_D, 1),
        b.reshape(1, _D),
        maskval,
    )
    e = e.reshape(n)

    partials = pl.kernel(
        _sc_denom_partials,
        out_type=jax.ShapeDtypeStruct((_NW * _NSEG,), jnp.float32),
        mesh=_mesh(),
        compiler_params=_SC_PARAMS,
        scratch_types=[
            pltpu.VMEM((_CHUNK,), jnp.float32),
            pltpu.VMEM((_CHUNK,), jnp.int32),
            pltpu.VMEM((_NSEG + _VL,), jnp.float32),
        ],
    )(e, index)

    aw, den = pl.kernel(
        _sc_aw,
        out_type=(
            jax.ShapeDtypeStruct((n,), jnp.float32),
            jax.ShapeDtypeStruct((_NSEG,), jnp.float32),
        ),
        mesh=_mesh(),
        compiler_params=_SC_PARAMS,
        scratch_types=[
            pltpu.VMEM((_NSEG,), jnp.float32),
            pltpu.VMEM((_NSEG,), jnp.float32),
            pltpu.VMEM((_CHUNK,), jnp.float32),
            pltpu.VMEM((_CHUNK,), jnp.int32),
            pltpu.VMEM((_CHUNK,), jnp.float32),
        ],
    )(partials, e, index)

    pooled = pl.kernel(
        _sc_wsum,
        out_type=jax.ShapeDtypeStruct((_NC * _NSEG, _D), jnp.float32),
        mesh=_mesh(),
        compiler_params=_SC_PARAMS,
        scratch_types=[
            pltpu.VMEM((_SUBR * _D,), jnp.float32),
            pltpu.VMEM((_PIECE,), jnp.float32),
            pltpu.VMEM((_PIECE,), jnp.int32),
            pltpu.VMEM((_WSEG, _D), jnp.float32),
            pltpu.VMEM((128,), jnp.int32),
            pltpu.VMEM((128,), jnp.int32),
            pltpu.VMEM_SHARED((_NSEG + 1, _D), jnp.float32),
        ],
    )(input.reshape(n * _D), e, index)

    out = pl.pallas_call(
        _tc3_kernel,
        grid=(5,),
        in_specs=[
            pl.BlockSpec((2, 2000, _D), lambda i: (0, i, 0)),
            pl.BlockSpec((2000, 1), lambda i: (i, 0)),
        ],
        out_specs=pl.BlockSpec((2000, _D), lambda i: (i, 0)),
        out_shape=jax.ShapeDtypeStruct((_NSEG, _D), jnp.float32),
    )(pooled.reshape(_NC, _NSEG, _D), den.reshape(_NSEG, 1))

    return (out, aw)
